# Initial kernel scaffold; baseline (speedup 1.0000x reference)
import jax
import jax.numpy as jnp
from jax.experimental import pallas as pl


def kernel(x, gate_w, gate_b, expert_w, expert_b, shared_w, shared_b):
    raise NotImplementedError("write your pallas kernel here")


# speedup vs baseline: 2.0998x; 2.0998x over previous
import jax
import jax.numpy as jnp
from jax.experimental import pallas as pl
from jax.experimental.pallas import tpu as pltpu

_HIDDEN = 1024
_E = 8
_BT = 1024


def _moe_tile(x_ref, xb_ref, gw_ref, gb_ref, ew_ref, eb_ref, sw_ref, sb_ref,
              out_ref, logits_ref):
    x = x_ref[...]
    xb = xb_ref[...]

    logits = jnp.dot(x, gw_ref[...]) + gb_ref[...]
    logits_ref[...] = logits
    probs = jax.nn.softmax(logits, axis=-1)

    iota = jax.lax.broadcasted_iota(jnp.int32, probs.shape, 1)
    v1 = jnp.max(probs, axis=-1, keepdims=True)
    i1 = jnp.min(jnp.where(probs == v1, iota, _E), axis=-1, keepdims=True)
    one1 = iota == i1
    probs2 = jnp.where(one1, -jnp.inf, probs)
    v2 = jnp.max(probs2, axis=-1, keepdims=True)
    i2 = jnp.min(jnp.where(probs2 == v2, iota, _E), axis=-1, keepdims=True)
    one2 = iota == i2
    denom = v1 + v2
    combine = jnp.where(one1, v1 / denom, 0.0) + jnp.where(one2, v2 / denom, 0.0)
    combine = combine.astype(jnp.float32)

    acc = jnp.dot(combine, eb_ref[...], preferred_element_type=jnp.float32)
    acc += sb_ref[0:1, :] + sb_ref[1:2, :]

    sw = (sw_ref[0] + sw_ref[1]).astype(jnp.bfloat16)
    acc += jnp.dot(xb, sw, preferred_element_type=jnp.float32)

    for e in range(_E):
        y = jnp.dot(xb, ew_ref[e], preferred_element_type=jnp.float32)
        acc += combine[:, e:e + 1] * y

    out_ref[...] = acc


def kernel(x, gate_w, gate_b, expert_w, expert_b, shared_w, shared_b):
    b, s, h = x.shape
    hs = x.reshape(-1, h)
    t = hs.shape[0]
    hs_bf = hs.astype(jnp.bfloat16)
    ew_bf = expert_w.astype(jnp.bfloat16)
    sw_bf = shared_w.astype(jnp.bfloat16)

    grid = (t // _BT,)
    out, logits = pl.pallas_call(
        _moe_tile,
        grid=grid,
        in_specs=[
            pl.BlockSpec((_BT, h), lambda i: (i, 0)),
            pl.BlockSpec((_BT, h), lambda i: (i, 0)),
            pl.BlockSpec((h, _E), lambda i: (0, 0)),
            pl.BlockSpec((1, _E), lambda i: (0, 0)),
            pl.BlockSpec((_E, h, h), lambda i: (0, 0, 0)),
            pl.BlockSpec((_E, h), lambda i: (0, 0)),
            pl.BlockSpec((2, h, h), lambda i: (0, 0, 0)),
            pl.BlockSpec((2, h), lambda i: (0, 0)),
        ],
        out_specs=[
            pl.BlockSpec((_BT, h), lambda i: (i, 0)),
            pl.BlockSpec((_BT, _E), lambda i: (i, 0)),
        ],
        out_shape=[
            jax.ShapeDtypeStruct((t, h), jnp.float32),
            jax.ShapeDtypeStruct((t, _E), jnp.float32),
        ],
        compiler_params=pltpu.CompilerParams(
            dimension_semantics=("arbitrary",),
        ),
    )(hs, hs_bf, gate_w, gate_b.reshape(1, _E), ew_bf, expert_b, sw_bf,
      shared_b)
    return out.reshape(b, s, h), logits
